# Initial kernel scaffold; baseline (speedup 1.0000x reference)
import jax
import jax.numpy as jnp
from jax.experimental import pallas as pl


def kernel(latent, W):
    raise NotImplementedError("write your pallas kernel here")


# speedup vs baseline: 2.0523x; 2.0523x over previous
import functools

import jax
import jax.numpy as jnp
from jax import lax
from jax.experimental import pallas as pl
from jax.experimental.pallas import tpu as pltpu

N_TOKENS = 8192
N_CODES = 8192
DIM = 32
CHUNK = 256
NSTEPS = N_TOKENS // CHUNK
BETA_C = 0.25

_PREC = lax.Precision.DEFAULT
_DN = (((1,), (1,)), ((), ()))
_DN_ROW = (((1,), (0,)), ((), ()))


def _vq_main(lat_ref, w_ref, idx_ref, cnt_ref, q_ref,
             summax_ref, dsum_ref, dmin_ref, wn_ref, sq_ref):
    i = pl.program_id(0)
    w = w_ref[...]

    @pl.when(i == 0)
    def _init():
        ww = w * w
        n = jnp.sqrt(jnp.sum(ww, axis=1, keepdims=True))
        wn_ref[...] = w / jnp.maximum(n, 1e-12)
        ones_row = jnp.ones((1, DIM), jnp.float32)
        sq_ref[...] = lax.dot_general(ones_row, ww, _DN,
                                      precision=lax.Precision.HIGHEST,
                                      preferred_element_type=jnp.float32)
        cnt_ref[...] = jnp.zeros((1, N_CODES), jnp.float32)
        summax_ref[0, 0] = jnp.float32(0.0)
        dsum_ref[0, 0] = jnp.float32(0.0)
        dmin_ref[0, 0] = jnp.float32(jnp.inf)

    lat = lat_ref[...]
    ln = lat / jnp.maximum(
        jnp.sqrt(jnp.sum(lat * lat, axis=1, keepdims=True)), 1e-12)
    cos = lax.dot_general(ln, wn_ref[...], _DN, precision=_PREC,
                          preferred_element_type=jnp.float32)
    m = jnp.max(cos, axis=1, keepdims=True)
    colids = lax.broadcasted_iota(jnp.int32, (CHUNK, N_CODES), 1)
    idx = jnp.min(jnp.where(cos == m, colids, N_CODES), axis=1)
    idx = idx.astype(jnp.int32)
    idx_ref[...] = idx.reshape(1, 1, CHUNK)
    summax_ref[0, 0] += jnp.sum(m)

    onehot = (colids == idx[:, None]).astype(jnp.float32)
    cnt_ref[...] += jnp.sum(onehot, axis=0, keepdims=True)
    q_ref[...] = lax.dot_general(onehot, w, _DN_ROW, precision=_PREC,
                                 preferred_element_type=jnp.float32)

    wc = w_ref[pl.ds(i * CHUNK, CHUNK), :]
    g = lax.dot_general(wc, w, _DN, precision=_PREC,
                        preferred_element_type=jnp.float32)
    sqc = jnp.sum(wc * wc, axis=1, keepdims=True)
    d2 = sqc + sq_ref[...] - 2.0 * g
    dist = jnp.sqrt(jnp.maximum(d2, 0.0))
    rowids = lax.broadcasted_iota(jnp.int32, (CHUNK, N_CODES), 0) + i * CHUNK
    offdiag = colids != rowids
    dsum_ref[0, 0] += jnp.sum(jnp.where(offdiag, dist, 0.0))
    dmin_ref[0, 0] = jnp.minimum(
        dmin_ref[0, 0], jnp.min(jnp.where(offdiag, dist, jnp.inf)))


def _vq_epilogue(lat_ref, q_ref, cnt_ref, summax_ref, dsum_ref, dmin_ref,
                 commit_ref, codebook_ref, perp_ref, sel_ref, avg_ref,
                 min_ref):
    diff = lat_ref[...] - q_ref[...]
    mse = jnp.sum(diff * diff) / jnp.float32(N_TOKENS * DIM)
    commit_ref[0, 0] = jnp.float32(BETA_C) * mse
    codebook_ref[0, 0] = mse
    p = cnt_ref[...] / jnp.float32(N_TOKENS)
    ent = -jnp.sum(p * jnp.log(p + 1e-10))
    perp_ref[0, 0] = jnp.exp(ent)
    sel_ref[0, 0] = summax_ref[0, 0] / jnp.float32(N_TOKENS)
    avg_ref[0, 0] = dsum_ref[0, 0] / jnp.float32(N_CODES * (N_CODES - 1))
    min_ref[0, 0] = dmin_ref[0, 0]


@jax.jit
def kernel(latent, W):
    B, S, D = latent.shape
    flat = latent.reshape(N_TOKENS, DIM)

    smem11 = pl.BlockSpec(memory_space=pltpu.SMEM)
    idx3, counts, qflat, summax, dsum, dmin = pl.pallas_call(
        _vq_main,
        grid=(NSTEPS,),
        in_specs=[
            pl.BlockSpec((CHUNK, DIM), lambda i: (i, 0)),
            pl.BlockSpec((N_CODES, DIM), lambda i: (0, 0)),
        ],
        out_specs=[
            pl.BlockSpec((1, 1, CHUNK), lambda i: (i, 0, 0)),
            pl.BlockSpec((1, N_CODES), lambda i: (0, 0)),
            pl.BlockSpec((CHUNK, DIM), lambda i: (i, 0)),
            smem11,
            smem11,
            smem11,
        ],
        out_shape=[
            jax.ShapeDtypeStruct((NSTEPS, 1, CHUNK), jnp.int32),
            jax.ShapeDtypeStruct((1, N_CODES), jnp.float32),
            jax.ShapeDtypeStruct((N_TOKENS, DIM), jnp.float32),
            jax.ShapeDtypeStruct((1, 1), jnp.float32),
            jax.ShapeDtypeStruct((1, 1), jnp.float32),
            jax.ShapeDtypeStruct((1, 1), jnp.float32),
        ],
        scratch_shapes=[
            pltpu.VMEM((N_CODES, DIM), jnp.float32),
            pltpu.VMEM((1, N_CODES), jnp.float32),
        ],
        compiler_params=pltpu.CompilerParams(
            dimension_semantics=("arbitrary",)),
    )(flat, W)

    commit, codebook, perp, sel, avg, mind = pl.pallas_call(
        _vq_epilogue,
        in_specs=[
            pl.BlockSpec((N_TOKENS, DIM), lambda: (0, 0)),
            pl.BlockSpec((N_TOKENS, DIM), lambda: (0, 0)),
            pl.BlockSpec((1, N_CODES), lambda: (0, 0)),
            smem11,
            smem11,
            smem11,
        ],
        out_specs=[smem11] * 6,
        out_shape=[jax.ShapeDtypeStruct((1, 1), jnp.float32)] * 6,
    )(flat, qflat, counts, summax, dsum, dmin)

    indices = idx3.reshape(N_TOKENS)
    quantized_st = qflat.reshape(B, S, D)
    return (quantized_st, indices, commit[0, 0], codebook[0, 0],
            perp[0, 0], sel[0, 0], avg[0, 0], mind[0, 0])


# speedup vs baseline: 2.3958x; 1.1674x over previous
import functools

import jax
import jax.numpy as jnp
from jax import lax
from jax.experimental import pallas as pl
from jax.experimental.pallas import tpu as pltpu

N_TOKENS = 8192
N_CODES = 8192
DIM = 32
CHUNK = 256
NSTEPS = N_TOKENS // CHUNK
BETA_C = 0.25

_PREC = lax.Precision.DEFAULT
_DN = (((1,), (1,)), ((), ()))
_DN_ROW = (((1,), (0,)), ((), ()))


def _vq_prep(w_ref, wn_ref, sq_ref):
    w = w_ref[...]
    ww = w * w
    n = jnp.sqrt(jnp.sum(ww, axis=1, keepdims=True))
    wn_ref[...] = w / jnp.maximum(n, 1e-12)
    ones_row = jnp.ones((1, DIM), jnp.float32)
    sq_ref[...] = lax.dot_general(ones_row, ww, _DN,
                                  precision=lax.Precision.HIGHEST,
                                  preferred_element_type=jnp.float32)


def _vq_main(lat_ref, w_ref, wn_ref, sq_ref, idx_ref, cnt_ref, q_ref,
             summax_ref, dsum_ref, dmin_ref):
    i = pl.program_id(0)
    w = w_ref[...]

    @pl.when(i == 0)
    def _init():
        cnt_ref[...] = jnp.zeros((1, N_CODES), jnp.float32)
        summax_ref[0, 0] = jnp.float32(0.0)
        dsum_ref[0, 0] = jnp.float32(0.0)
        dmin_ref[0, 0] = jnp.float32(jnp.inf)

    lat = lat_ref[...]
    ln = lat / jnp.maximum(
        jnp.sqrt(jnp.sum(lat * lat, axis=1, keepdims=True)), 1e-12)
    cos = lax.dot_general(ln, wn_ref[...], _DN, precision=_PREC,
                          preferred_element_type=jnp.float32)
    m = jnp.max(cos, axis=1, keepdims=True)
    colids = lax.broadcasted_iota(jnp.int32, (CHUNK, N_CODES), 1)
    idx = jnp.min(jnp.where(cos == m, colids, N_CODES), axis=1)
    idx = idx.astype(jnp.int32)
    idx_ref[...] = idx.reshape(1, 1, CHUNK)
    summax_ref[0, 0] += jnp.sum(m)

    onehot = (colids == idx[:, None]).astype(jnp.float32)
    cnt_ref[...] += jnp.sum(onehot, axis=0, keepdims=True)
    q_ref[...] = lax.dot_general(onehot, w, _DN_ROW, precision=_PREC,
                                 preferred_element_type=jnp.float32)

    wc = w_ref[pl.ds(i * CHUNK, CHUNK), :]
    g = lax.dot_general(wc, w, _DN, precision=_PREC,
                        preferred_element_type=jnp.float32)
    sqc = jnp.sum(wc * wc, axis=1, keepdims=True)
    d2 = jnp.maximum(sqc + sq_ref[...] - 2.0 * g, 0.0)
    dist = d2 * lax.rsqrt(jnp.maximum(d2, 1e-30))
    dsum_ref[0, 0] += jnp.sum(dist)
    rowids = lax.broadcasted_iota(jnp.int32, (CHUNK, N_CODES), 0) + i * CHUNK
    offdiag = colids != rowids
    dmin_ref[0, 0] = jnp.minimum(
        dmin_ref[0, 0], jnp.min(jnp.where(offdiag, d2, jnp.inf)))


def _vq_epilogue(lat_ref, q_ref, cnt_ref, summax_ref, dsum_ref, dmin_ref,
                 commit_ref, codebook_ref, perp_ref, sel_ref, avg_ref,
                 min_ref):
    diff = lat_ref[...] - q_ref[...]
    mse = jnp.sum(diff * diff) / jnp.float32(N_TOKENS * DIM)
    commit_ref[0, 0] = jnp.float32(BETA_C) * mse
    codebook_ref[0, 0] = mse
    p = cnt_ref[...] / jnp.float32(N_TOKENS)
    ent = -jnp.sum(p * jnp.log(p + 1e-10))
    perp_ref[0, 0] = jnp.exp(ent)
    sel_ref[0, 0] = summax_ref[0, 0] / jnp.float32(N_TOKENS)
    avg_ref[0, 0] = dsum_ref[0, 0] / jnp.float32(N_CODES * (N_CODES - 1))
    min_ref[0, 0] = jnp.sqrt(jnp.maximum(dmin_ref[0, 0], 0.0))


@jax.jit
def kernel(latent, W):
    B, S, D = latent.shape
    flat = latent.reshape(N_TOKENS, DIM)

    smem11 = pl.BlockSpec(memory_space=pltpu.SMEM)
    wn, sq = pl.pallas_call(
        _vq_prep,
        in_specs=[pl.BlockSpec((N_CODES, DIM), lambda: (0, 0))],
        out_specs=[
            pl.BlockSpec((N_CODES, DIM), lambda: (0, 0)),
            pl.BlockSpec((1, N_CODES), lambda: (0, 0)),
        ],
        out_shape=[
            jax.ShapeDtypeStruct((N_CODES, DIM), jnp.float32),
            jax.ShapeDtypeStruct((1, N_CODES), jnp.float32),
        ],
    )(W)

    idx3, counts, qflat, summax, dsum, dmin = pl.pallas_call(
        _vq_main,
        grid=(NSTEPS,),
        in_specs=[
            pl.BlockSpec((CHUNK, DIM), lambda i: (i, 0)),
            pl.BlockSpec((N_CODES, DIM), lambda i: (0, 0)),
            pl.BlockSpec((N_CODES, DIM), lambda i: (0, 0)),
            pl.BlockSpec((1, N_CODES), lambda i: (0, 0)),
        ],
        out_specs=[
            pl.BlockSpec((1, 1, CHUNK), lambda i: (i, 0, 0)),
            pl.BlockSpec((1, N_CODES), lambda i: (0, 0)),
            pl.BlockSpec((CHUNK, DIM), lambda i: (i, 0)),
            smem11,
            smem11,
            smem11,
        ],
        out_shape=[
            jax.ShapeDtypeStruct((NSTEPS, 1, CHUNK), jnp.int32),
            jax.ShapeDtypeStruct((1, N_CODES), jnp.float32),
            jax.ShapeDtypeStruct((N_TOKENS, DIM), jnp.float32),
            jax.ShapeDtypeStruct((1, 1), jnp.float32),
            jax.ShapeDtypeStruct((1, 1), jnp.float32),
            jax.ShapeDtypeStruct((1, 1), jnp.float32),
        ],
        compiler_params=pltpu.CompilerParams(
            dimension_semantics=("arbitrary",)),
    )(flat, W, wn, sq)

    commit, codebook, perp, sel, avg, mind = pl.pallas_call(
        _vq_epilogue,
        in_specs=[
            pl.BlockSpec((N_TOKENS, DIM), lambda: (0, 0)),
            pl.BlockSpec((N_TOKENS, DIM), lambda: (0, 0)),
            pl.BlockSpec((1, N_CODES), lambda: (0, 0)),
            smem11,
            smem11,
            smem11,
        ],
        out_specs=[smem11] * 6,
        out_shape=[jax.ShapeDtypeStruct((1, 1), jnp.float32)] * 6,
    )(flat, qflat, counts, summax, dsum, dmin)

    indices = idx3.reshape(N_TOKENS)
    quantized_st = qflat.reshape(B, S, D)
    return (quantized_st, indices, commit[0, 0], codebook[0, 0],
            perp[0, 0], sel[0, 0], avg[0, 0], mind[0, 0])
